# baseline (speedup 1.0000x reference)
import functools

import jax
import jax.numpy as jnp
from jax import lax
from jax.experimental import pallas as pl
from jax.experimental.pallas import tpu as pltpu
from jax.experimental.pallas import tpu_sc as plsc

NUM_ROWS = 1000000
DIM = 16
BATCH = 16384 * 26

NUM_CORES = 2
NUM_SUBCORES = 16
NUM_WORKERS = NUM_CORES * NUM_SUBCORES
B_PER_W = BATCH // NUM_WORKERS
CHUNK = 1664
NCHUNK = B_PER_W // CHUNK

_mesh = plsc.VectorSubcoreMesh(
    core_axis_name="c", subcore_axis_name="s",
    num_cores=NUM_CORES, num_subcores=NUM_SUBCORES)


@functools.partial(
    pl.kernel,
    out_type=jax.ShapeDtypeStruct((BATCH, DIM), jnp.float32),
    mesh=_mesh,
    scratch_types=[
        pltpu.VMEM((CHUNK,), jnp.int32),
        pltpu.VMEM((CHUNK, DIM), jnp.float32),
        pltpu.SemaphoreType.DMA,
    ],
    compiler_params=pltpu.CompilerParams(use_tc_tiling_on_sc=False),
)
def _gather_kernel(x_hbm, table_hbm, out_hbm, idx_v, rows_v, sem):
    wid = lax.axis_index("s") * NUM_CORES + lax.axis_index("c")
    base = wid * B_PER_W

    def chunk_body(j, carry):
        off = base + j * CHUNK
        pltpu.sync_copy(x_hbm.at[pl.ds(off, CHUNK)], idx_v)
        pltpu.async_copy(table_hbm.at[idx_v], rows_v, sem).wait()
        pltpu.sync_copy(rows_v, out_hbm.at[pl.ds(off, CHUNK)])
        return carry

    lax.fori_loop(0, NCHUNK, chunk_body, 0)


def kernel(x, table):
    flat = x.reshape(-1)
    out = _gather_kernel(flat, table)
    return out.reshape(x.shape + (DIM,))


# speedup vs baseline: 1.0096x; 1.0096x over previous
import functools

import jax
import jax.numpy as jnp
from jax import lax
from jax.experimental import pallas as pl
from jax.experimental.pallas import tpu as pltpu
from jax.experimental.pallas import tpu_sc as plsc

NUM_ROWS = 1000000
DIM = 16
BATCH = 16384 * 26

NUM_CORES = 2
NUM_SUBCORES = 16
NUM_WORKERS = NUM_CORES * NUM_SUBCORES
B_PER_W = BATCH // NUM_WORKERS
CHUNK = 1664
NCHUNK = B_PER_W // CHUNK
NBUF_I = 2
NBUF_R = 3

_mesh = plsc.VectorSubcoreMesh(
    core_axis_name="c", subcore_axis_name="s",
    num_cores=NUM_CORES, num_subcores=NUM_SUBCORES)


@functools.partial(
    pl.kernel,
    out_type=jax.ShapeDtypeStruct((BATCH, DIM), jnp.float32),
    mesh=_mesh,
    scratch_types=(
        [pltpu.VMEM((CHUNK,), jnp.int32) for _ in range(NBUF_I)]
        + [pltpu.VMEM((CHUNK, DIM), jnp.float32) for _ in range(NBUF_R)]
        + [pltpu.SemaphoreType.DMA for _ in range(NBUF_I + 2 * NBUF_R)]
    ),
    compiler_params=pltpu.CompilerParams(use_tc_tiling_on_sc=False),
)
def _gather_kernel(x_hbm, table_hbm, out_hbm, *scratch):
    idx_v = scratch[:NBUF_I]
    rows_v = scratch[NBUF_I:NBUF_I + NBUF_R]
    sems = scratch[NBUF_I + NBUF_R:]
    sem_i = sems[:NBUF_I]
    sem_g = sems[NBUF_I:NBUF_I + NBUF_R]
    sem_o = sems[NBUF_I + NBUF_R:]

    wid = lax.axis_index("s") * NUM_CORES + lax.axis_index("c")
    base = wid * B_PER_W

    def copy_idx(j):
        return pltpu.async_copy(
            x_hbm.at[pl.ds(base + j * CHUNK, CHUNK)],
            idx_v[j % NBUF_I], sem_i[j % NBUF_I])

    def gather(j):
        return pltpu.async_copy(
            table_hbm.at[idx_v[j % NBUF_I]],
            rows_v[j % NBUF_R], sem_g[j % NBUF_R])

    def copy_out(j):
        return pltpu.async_copy(
            rows_v[j % NBUF_R],
            out_hbm.at[pl.ds(base + j * CHUNK, CHUNK)],
            sem_o[j % NBUF_R])

    h_idx = [None] * NCHUNK
    h_g = [None] * NCHUNK
    h_out = [None] * NCHUNK

    for j in range(min(NBUF_I, NCHUNK)):
        h_idx[j] = copy_idx(j)
    h_idx[0].wait()
    h_g[0] = gather(0)

    for j in range(NCHUNK):
        if j + 1 < NCHUNK:
            if j + 1 >= NBUF_R:
                h_out[j + 1 - NBUF_R].wait()
            h_idx[j + 1].wait()
            h_g[j + 1] = gather(j + 1)
        h_g[j].wait()
        h_out[j] = copy_out(j)
        if j + NBUF_I < NCHUNK:
            h_idx[j + NBUF_I] = copy_idx(j + NBUF_I)

    for j in range(max(0, NCHUNK - NBUF_R), NCHUNK):
        h_out[j].wait()


def kernel(x, table):
    flat = x.reshape(-1)
    out = _gather_kernel(flat, table)
    return out.reshape(x.shape + (DIM,))
